# Initial kernel scaffold; baseline (speedup 1.0000x reference)
import jax
import jax.numpy as jnp
from jax.experimental import pallas as pl


def kernel(x, dynamic_indices, ro_W, ro_b, gamma, beta):
    raise NotImplementedError("write your pallas kernel here")


# speedup vs baseline: 4.5821x; 4.5821x over previous
import functools

import jax
import jax.numpy as jnp
from jax import lax
from jax.experimental import pallas as pl
from jax.experimental.pallas import tpu as pltpu
from jax.experimental.pallas import tpu_sc as plsc

B, N, C, K = 8, 10000, 128, 9
G = 4
EPS = 1e-5

NW = 32
CHUNK = 320
N_PAD = NW * CHUNK
NB = 64
NBLK = CHUNK // NB


def _proj_body(x_ref, w_ref, b_ref, o_ref):
    xb = x_ref[0]
    p = jnp.dot(xb, w_ref[...], preferred_element_type=jnp.float32)
    o_ref[...] = (p + b_ref[0]).reshape(1, N, 1)


def _project(x, ro_W, ro_b):
    return pl.pallas_call(
        _proj_body,
        grid=(B,),
        in_specs=[
            pl.BlockSpec((1, N, C), lambda b: (b, 0, 0)),
            pl.BlockSpec((C, 1), lambda b: (0, 0)),
            pl.BlockSpec((1,), lambda b: (0,)),
        ],
        out_specs=pl.BlockSpec((1, N, 1), lambda b: (b, 0, 0)),
        out_shape=jax.ShapeDtypeStruct((B, N, 1), jnp.float32),
    )(x, ro_W, ro_b)


def _sc_pool_body(x_hbm, idx_hbm, p_hbm, out_hbm,
                  idxr, idxb, pv, rows, wbuf, outb, sem):
    wid = lax.axis_index("s") * 2 + lax.axis_index("c")
    pltpu.sync_copy(idx_hbm.at[wid], idxr)

    def batch_body(b, _):
        pltpu.sync_copy(p_hbm.at[b], pv)
        off = b * N
        for k in range(K):
            for blk in range(NBLK):
                for j in range(NB // 16):
                    sl = pl.ds(j * 16, 16)
                    idxb[k, blk, sl] = idxr[k, blk, sl] + off

        for blk in range(NBLK):
            copies = [
                pltpu.async_copy(x_hbm.at[idxb.at[k, blk]], rows.at[k], sem)
                for k in range(K)
            ]
            for j in range(NB // 16):
                sl = pl.ds(j * 16, 16)
                acc = plsc.load_gather(pv, [idxr[0, blk, sl]])
                for k in range(1, K):
                    acc = acc + plsc.load_gather(pv, [idxr[k, blk, sl]])
                s = jnp.abs(acc * (1.0 / K))
                s = jnp.minimum(s * K, float(K - 1))
                for k in range(K):
                    wbuf[k, sl] = jnp.clip(s - float(k) + 1.0, 0.0, 1.0)
            for cp in copies:
                cp.wait()

            def node_body(n, _):
                accs = [jnp.zeros((16,), jnp.float32) for _ in range(C // 16)]
                for k in range(K):
                    w = wbuf[k, pl.ds(n, 16)][0]
                    for v in range(C // 16):
                        accs[v] = accs[v] + w * rows[k, n, pl.ds(v * 16, 16)]
                for v in range(C // 16):
                    outb[n, pl.ds(v * 16, 16)] = accs[v]
                return 0

            lax.fori_loop(0, NB, node_body, 0)
            node0 = wid * CHUNK + blk * NB
            pltpu.sync_copy(outb, out_hbm.at[b, pl.ds(node0, NB)])
        return 0

    lax.fori_loop(0, B, batch_body, 0)


def _sc_pool(x2d, idx_w, p):
    mesh = plsc.VectorSubcoreMesh(core_axis_name="c", subcore_axis_name="s")
    kern = functools.partial(
        pl.kernel,
        mesh=mesh,
        compiler_params=pltpu.CompilerParams(
            needs_layout_passes=False, use_tc_tiling_on_sc=False),
        out_type=jax.ShapeDtypeStruct((B, N_PAD, C), jnp.float32),
        scratch_types=[
            pltpu.VMEM((K, NBLK, NB), jnp.int32),
            pltpu.VMEM((K, NBLK, NB), jnp.int32),
            pltpu.VMEM((N,), jnp.float32),
            pltpu.VMEM((K, NB, C), jnp.float32),
            pltpu.VMEM((K, NB + 16), jnp.float32),
            pltpu.VMEM((NB, C), jnp.float32),
            pltpu.SemaphoreType.DMA,
        ],
    )(_sc_pool_body)
    return kern(x2d, idx_w, p)


def _gn_body(y_ref, g_ref, b_ref, o_ref):
    y = y_ref[0, :N, :]
    s_ch = jnp.sum(y, axis=0, keepdims=True)
    q_ch = jnp.sum(y * y, axis=0, keepdims=True)
    gi = lax.broadcasted_iota(jnp.int32, (C, C), 0) // (C // G)
    gj = lax.broadcasted_iota(jnp.int32, (C, C), 1) // (C // G)
    M = jnp.where(gi == gj, 1.0 / ((C // G) * N), 0.0).astype(jnp.float32)
    mean_c = jnp.dot(s_ch, M, preferred_element_type=jnp.float32)
    ex2_c = jnp.dot(q_ch, M, preferred_element_type=jnp.float32)
    var_c = ex2_c - mean_c * mean_c
    rstd_c = lax.rsqrt(var_c + EPS)
    gam = g_ref[...].reshape(1, C)
    bet = b_ref[...].reshape(1, C)
    o_ref[...] = ((y - mean_c) * (rstd_c * gam) + bet).reshape(1, N, C)


def _group_norm(pool_pad, gamma, beta):
    return pl.pallas_call(
        _gn_body,
        grid=(B,),
        in_specs=[
            pl.BlockSpec((1, N_PAD, C), lambda b: (b, 0, 0)),
            pl.BlockSpec((C,), lambda b: (0,)),
            pl.BlockSpec((C,), lambda b: (0,)),
        ],
        out_specs=pl.BlockSpec((1, N, C), lambda b: (b, 0, 0)),
        out_shape=jax.ShapeDtypeStruct((B, N, C), jnp.float32),
    )(pool_pad, gamma, beta)


def kernel(x, dynamic_indices, ro_W, ro_b, gamma, beta):
    p3 = _project(x, ro_W, ro_b)
    p = p3.reshape(B, N)

    idx_t = jnp.pad(dynamic_indices, ((0, N_PAD - N), (0, 0))).T
    idx_w = (idx_t.reshape(K, NW, NBLK, NB)
             .transpose(1, 0, 2, 3))
    x2d = x.reshape(B * N, C)

    pool_pad = _sc_pool(x2d, idx_w, p)
    return _group_norm(pool_pad, gamma, beta)


# speedup vs baseline: 9.3209x; 2.0342x over previous
import functools

import jax
import jax.numpy as jnp
from jax import lax
from jax.experimental import pallas as pl
from jax.experimental.pallas import tpu as pltpu
from jax.experimental.pallas import tpu_sc as plsc

B, N, C, K = 8, 10000, 128, 9
G = 4
EPS = 1e-5

NW = 32
CHUNK = 320
N_PAD = NW * CHUNK
NB = 8
NBLK = CHUNK // NB
V16 = C // 16


def _proj_body(x_ref, w_ref, b_ref, p_ref, xt_ref):
    xb = x_ref[0]
    p = jnp.dot(xb, w_ref[...], preferred_element_type=jnp.float32)
    p_ref[...] = (p + b_ref[0]).reshape(1, N, 1)
    xt_ref[...] = xb


def _project(x, ro_W, ro_b):
    return pl.pallas_call(
        _proj_body,
        grid=(B,),
        in_specs=[
            pl.BlockSpec((1, N, C), lambda b: (b, 0, 0)),
            pl.BlockSpec((C, 1), lambda b: (0, 0)),
            pl.BlockSpec((1,), lambda b: (0,)),
        ],
        out_specs=[
            pl.BlockSpec((1, N, 1), lambda b: (b, 0, 0)),
            pl.BlockSpec((N, C), lambda b: (0, b)),
        ],
        out_shape=[
            jax.ShapeDtypeStruct((B, N, 1), jnp.float32),
            jax.ShapeDtypeStruct((N, B * C), jnp.float32),
        ],
    )(x, ro_W, ro_b)


def _sc_pool_body(xt_hbm, idx_hbm, p_hbm, out_hbm,
                  idxs, pv, wbuf, rows, outb, sem):
    wid = lax.axis_index("s") * 2 + lax.axis_index("c")
    pltpu.sync_copy(idx_hbm.at[wid], idxs)
    node_base = wid * CHUNK

    def weights_batch(b, _):
        pltpu.sync_copy(p_hbm.at[b], pv)

        def weights_grp(j, _):
            sl = pl.ds(j * 16, 16)
            acc = plsc.load_gather(pv, [idxs[0, sl]])
            for k in range(1, K):
                acc = acc + plsc.load_gather(pv, [idxs[k, sl]])
            s = jnp.minimum(jnp.abs(acc * (1.0 / K)) * K, float(K - 1))
            for k in range(K):
                wbuf[b, k, sl] = jnp.clip(s - float(k) + 1.0, 0.0, 1.0)
            return 0

        lax.fori_loop(0, CHUNK // 16, weights_grp, 0)
        return 0

    lax.fori_loop(0, B, weights_batch, 0)

    def blk_body(blk, _):
        copies = [
            pltpu.async_copy(xt_hbm.at[idxs.at[k, pl.ds(blk * NB, NB)]],
                             rows.at[k], sem)
            for k in range(K)
        ]
        for cp in copies:
            cp.wait()

        for n in range(NB):
            nloc = blk * NB + n

            def acc_batch(b, _):
                ws = [wbuf[b, k, pl.ds(nloc, 16)][0] for k in range(K)]
                for v in range(V16):
                    sl = pl.ds(b * C + v * 16, 16)
                    acc = ws[0] * rows[0, n, sl]
                    for k in range(1, K):
                        acc = acc + ws[k] * rows[k, n, sl]
                    outb[n, sl] = acc
                return 0

            lax.fori_loop(0, B, acc_batch, 0)

        pltpu.sync_copy(outb, out_hbm.at[pl.ds(node_base + blk * NB, NB)])
        return 0

    lax.fori_loop(0, NBLK, blk_body, 0)


def _sc_pool(xt, idx_w, p):
    mesh = plsc.VectorSubcoreMesh(core_axis_name="c", subcore_axis_name="s")
    kern = functools.partial(
        pl.kernel,
        mesh=mesh,
        compiler_params=pltpu.CompilerParams(
            needs_layout_passes=False, use_tc_tiling_on_sc=False),
        out_type=jax.ShapeDtypeStruct((N_PAD, B * C), jnp.float32),
        scratch_types=[
            pltpu.VMEM((K, CHUNK), jnp.int32),
            pltpu.VMEM((N,), jnp.float32),
            pltpu.VMEM((B, K, CHUNK + 16), jnp.float32),
            pltpu.VMEM((K, NB, B * C), jnp.float32),
            pltpu.VMEM((NB, B * C), jnp.float32),
            pltpu.SemaphoreType.DMA,
        ],
    )(_sc_pool_body)
    return kern(xt, idx_w, p)


def _gn_body(y_ref, g_ref, b_ref, o_ref):
    y = y_ref[:N, :]
    s_ch = jnp.sum(y, axis=0, keepdims=True)
    q_ch = jnp.sum(y * y, axis=0, keepdims=True)
    gi = lax.broadcasted_iota(jnp.int32, (C, C), 0) // (C // G)
    gj = lax.broadcasted_iota(jnp.int32, (C, C), 1) // (C // G)
    M = jnp.where(gi == gj, 1.0 / ((C // G) * N), 0.0).astype(jnp.float32)
    mean_c = jnp.dot(s_ch, M, preferred_element_type=jnp.float32)
    ex2_c = jnp.dot(q_ch, M, preferred_element_type=jnp.float32)
    var_c = ex2_c - mean_c * mean_c
    rstd_c = lax.rsqrt(var_c + EPS)
    gam = g_ref[...].reshape(1, C)
    bet = b_ref[...].reshape(1, C)
    o_ref[...] = ((y - mean_c) * (rstd_c * gam) + bet).reshape(1, N, C)


def _group_norm(pool_t, gamma, beta):
    return pl.pallas_call(
        _gn_body,
        grid=(B,),
        in_specs=[
            pl.BlockSpec((N_PAD, C), lambda b: (0, b)),
            pl.BlockSpec((C,), lambda b: (0,)),
            pl.BlockSpec((C,), lambda b: (0,)),
        ],
        out_specs=pl.BlockSpec((1, N, C), lambda b: (b, 0, 0)),
        out_shape=jax.ShapeDtypeStruct((B, N, C), jnp.float32),
    )(pool_t, gamma, beta)


def kernel(x, dynamic_indices, ro_W, ro_b, gamma, beta):
    p3, xt = _project(x, ro_W, ro_b)
    p = p3.reshape(B, N)

    idx_t = jnp.pad(dynamic_indices, ((0, N_PAD - N), (0, 0))).T
    idx_w = (idx_t.reshape(K, NW, CHUNK)
             .transpose(1, 0, 2))

    pool_t = _sc_pool(xt, idx_w, p)
    return _group_norm(pool_t, gamma, beta)


# speedup vs baseline: 10.7765x; 1.1562x over previous
import functools

import jax
import jax.numpy as jnp
from jax import lax
from jax.experimental import pallas as pl
from jax.experimental.pallas import tpu as pltpu
from jax.experimental.pallas import tpu_sc as plsc

B, N, C, K = 8, 10000, 128, 9
G = 4
EPS = 1e-5

NW = 32
CHUNK = 320
N_PAD = NW * CHUNK
NB = 8
NBLK = CHUNK // NB
V16 = C // 16


def _proj_body(x_ref, w_ref, b_ref, p_ref, xt_ref):
    xb = x_ref[0]
    p = jnp.dot(xb, w_ref[...], preferred_element_type=jnp.float32)
    p_ref[...] = (p + b_ref[0]).reshape(1, N, 1)
    xt_ref[...] = xb


def _project(x, ro_W, ro_b):
    return pl.pallas_call(
        _proj_body,
        grid=(B,),
        in_specs=[
            pl.BlockSpec((1, N, C), lambda b: (b, 0, 0)),
            pl.BlockSpec((C, 1), lambda b: (0, 0)),
            pl.BlockSpec((1,), lambda b: (0,)),
        ],
        out_specs=[
            pl.BlockSpec((1, N, 1), lambda b: (b, 0, 0)),
            pl.BlockSpec((N, C), lambda b: (0, b)),
        ],
        out_shape=[
            jax.ShapeDtypeStruct((B, N, 1), jnp.float32),
            jax.ShapeDtypeStruct((N, B * C), jnp.float32),
        ],
    )(x, ro_W, ro_b)


def _sc_pool_body(xt_hbm, idx_hbm, p_hbm, out_hbm,
                  idxr, idxs, pv, wbuf, rows, outb, sem):
    wid = lax.axis_index("s") * 2 + lax.axis_index("c")
    node_base = jnp.minimum(wid * CHUNK, N - CHUNK)
    pltpu.sync_copy(idx_hbm.at[pl.ds(node_base, CHUNK)], idxr)

    lane = lax.iota(jnp.int32, 16)
    for k in range(K):
        kvec = jnp.full((16,), k, jnp.int32)

        def tr_grp(j, _):
            rows16 = j * 16 + lane
            idxs[k, pl.ds(j * 16, 16)] = plsc.load_gather(idxr, [rows16, kvec])
            return 0

        lax.fori_loop(0, CHUNK // 16, tr_grp, 0)

    def weights_batch(b, _):
        pltpu.sync_copy(p_hbm.at[b], pv)

        def weights_grp(j, _):
            sl = pl.ds(j * 16, 16)
            acc = plsc.load_gather(pv, [idxs[0, sl]])
            for k in range(1, K):
                acc = acc + plsc.load_gather(pv, [idxs[k, sl]])
            s = jnp.minimum(jnp.abs(acc * (1.0 / K)) * K, float(K - 1))
            for k in range(K):
                wbuf[b, k, sl] = jnp.clip(s - float(k) + 1.0, 0.0, 1.0)
            return 0

        lax.fori_loop(0, CHUNK // 16, weights_grp, 0)
        return 0

    lax.fori_loop(0, B, weights_batch, 0)

    def blk_body(blk, _):
        copies = [
            pltpu.async_copy(xt_hbm.at[idxs.at[k, pl.ds(blk * NB, NB)]],
                             rows.at[k], sem)
            for k in range(K)
        ]
        for cp in copies:
            cp.wait()

        for n in range(NB):
            nloc = blk * NB + n

            def acc_batch(b, _):
                ws = [wbuf[b, k, pl.ds(nloc, 16)][0] for k in range(K)]
                for v in range(V16):
                    sl = pl.ds(b * C + v * 16, 16)
                    acc = ws[0] * rows[0, n, sl]
                    for k in range(1, K):
                        acc = acc + ws[k] * rows[k, n, sl]
                    outb[n, sl] = acc
                return 0

            lax.fori_loop(0, B, acc_batch, 0)

        pltpu.sync_copy(outb, out_hbm.at[pl.ds(node_base + blk * NB, NB)])
        return 0

    lax.fori_loop(0, NBLK, blk_body, 0)


def _sc_pool(xt, idx_w, p):
    mesh = plsc.VectorSubcoreMesh(core_axis_name="c", subcore_axis_name="s")
    kern = functools.partial(
        pl.kernel,
        mesh=mesh,
        compiler_params=pltpu.CompilerParams(
            needs_layout_passes=False, use_tc_tiling_on_sc=False),
        out_type=jax.ShapeDtypeStruct((N, B * C), jnp.float32),
        scratch_types=[
            pltpu.VMEM((CHUNK, K), jnp.int32),
            pltpu.VMEM((K, CHUNK), jnp.int32),
            pltpu.VMEM((N,), jnp.float32),
            pltpu.VMEM((B, K, CHUNK + 16), jnp.float32),
            pltpu.VMEM((K, NB, B * C), jnp.float32),
            pltpu.VMEM((NB, B * C), jnp.float32),
            pltpu.SemaphoreType.DMA,
        ],
    )(_sc_pool_body)
    return kern(xt, idx_w, p)


def _gn_body(y_ref, g_ref, b_ref, o_ref):
    y = y_ref[...]
    s_ch = jnp.sum(y, axis=0, keepdims=True)
    q_ch = jnp.sum(y * y, axis=0, keepdims=True)
    gi = lax.broadcasted_iota(jnp.int32, (C, C), 0) // (C // G)
    gj = lax.broadcasted_iota(jnp.int32, (C, C), 1) // (C // G)
    M = jnp.where(gi == gj, 1.0 / ((C // G) * N), 0.0).astype(jnp.float32)
    mean_c = jnp.dot(s_ch, M, preferred_element_type=jnp.float32)
    ex2_c = jnp.dot(q_ch, M, preferred_element_type=jnp.float32)
    var_c = ex2_c - mean_c * mean_c
    rstd_c = lax.rsqrt(var_c + EPS)
    gam = g_ref[...].reshape(1, C)
    bet = b_ref[...].reshape(1, C)
    o_ref[...] = ((y - mean_c) * (rstd_c * gam) + bet).reshape(1, N, C)


def _group_norm(pool_t, gamma, beta):
    return pl.pallas_call(
        _gn_body,
        grid=(B,),
        in_specs=[
            pl.BlockSpec((N, C), lambda b: (0, b)),
            pl.BlockSpec((C,), lambda b: (0,)),
            pl.BlockSpec((C,), lambda b: (0,)),
        ],
        out_specs=pl.BlockSpec((1, N, C), lambda b: (b, 0, 0)),
        out_shape=jax.ShapeDtypeStruct((B, N, C), jnp.float32),
    )(pool_t, gamma, beta)


def kernel(x, dynamic_indices, ro_W, ro_b, gamma, beta):
    p3, xt = _project(x, ro_W, ro_b)
    p = p3.reshape(B, N)
    pool_t = _sc_pool(xt, dynamic_indices, p)
    return _group_norm(pool_t, gamma, beta)


# speedup vs baseline: 15.4823x; 1.4367x over previous
import functools

import jax
import jax.numpy as jnp
from jax import lax
from jax.experimental import pallas as pl
from jax.experimental.pallas import tpu as pltpu
from jax.experimental.pallas import tpu_sc as plsc

B, N, C, K = 8, 10000, 128, 9
G = 4
EPS = 1e-5

NW = 32
CHUNK = 320
N_PAD = NW * CHUNK
NB = 8
NBLK = CHUNK // NB
V16 = C // 16
KA = 4


def _proj_body(x_ref, w_ref, b_ref, p_ref, xt_ref):
    xb = x_ref[0]
    p = jnp.dot(xb, w_ref[...], preferred_element_type=jnp.float32)
    p_ref[...] = (p + b_ref[0]).reshape(1, N, 1)
    xt_ref[...] = xb


def _project(x, ro_W, ro_b):
    return pl.pallas_call(
        _proj_body,
        grid=(B,),
        in_specs=[
            pl.BlockSpec((1, N, C), lambda b: (b, 0, 0)),
            pl.BlockSpec((C, 1), lambda b: (0, 0)),
            pl.BlockSpec((1,), lambda b: (0,)),
        ],
        out_specs=[
            pl.BlockSpec((1, N, 1), lambda b: (b, 0, 0)),
            pl.BlockSpec((N, C), lambda b: (0, b)),
        ],
        out_shape=[
            jax.ShapeDtypeStruct((B, N, 1), jnp.float32),
            jax.ShapeDtypeStruct((N, B * C), jnp.float32),
        ],
    )(x, ro_W, ro_b)


def _sc_pool_body(xt_hbm, idx_hbm, p_hbm, out_hbm,
                  idxr, idxs, pv, wbuf, rowsA, rowsB, outb, semA, semB):
    wid = lax.axis_index("s") * 2 + lax.axis_index("c")
    node_base = jnp.minimum(wid * CHUNK, N - CHUNK)
    pltpu.sync_copy(idx_hbm.at[pl.ds(node_base, CHUNK)], idxr)

    lane = lax.iota(jnp.int32, 16)
    for k in range(K):
        kvec = jnp.full((16,), k, jnp.int32)

        def tr_grp(j, _):
            rows16 = j * 16 + lane
            idxs[k, pl.ds(j * 16, 16)] = plsc.load_gather(idxr, [rows16, kvec])
            return 0

        lax.fori_loop(0, CHUNK // 16, tr_grp, 0)

    def weights_batch(b, _):
        pltpu.sync_copy(p_hbm.at[b], pv)

        def weights_grp(j, _):
            sl = pl.ds(j * 16, 16)
            acc = plsc.load_gather(pv, [idxs[0, sl]])
            for k in range(1, K):
                acc = acc + plsc.load_gather(pv, [idxs[k, sl]])
            s = jnp.minimum(jnp.abs(acc * (1.0 / K)) * K, float(K - 1))
            for k in range(K):
                wbuf[b, k, sl] = jnp.clip(s - float(k) + 1.0, 0.0, 1.0)
            return 0

        lax.fori_loop(0, CHUNK // 16, weights_grp, 0)
        return 0

    lax.fori_loop(0, B, weights_batch, 0)

    def fire(grp, blk):
        ks = range(KA) if grp == 0 else range(KA, K)
        buf, s = (rowsA, semA) if grp == 0 else (rowsB, semB)
        for k in ks:
            pltpu.async_copy(xt_hbm.at[idxs.at[k, pl.ds(blk * NB, NB)]],
                             buf.at[k if grp == 0 else k - KA], s)

    def drain(grp, blk):
        ks = range(KA) if grp == 0 else range(KA, K)
        buf, s = (rowsA, semA) if grp == 0 else (rowsB, semB)
        for k in ks:
            pltpu.make_async_copy(
                xt_hbm.at[idxs.at[k, pl.ds(blk * NB, NB)]],
                buf.at[k if grp == 0 else k - KA], s).wait()

    def accum(grp, blk):
        for n in range(NB):
            nloc = blk * NB + n

            def acc_batch(b, _):
                if grp == 0:
                    ws = [wbuf[b, k, pl.ds(nloc, 16)][0] for k in range(KA)]
                    for v in range(V16):
                        sl = pl.ds(b * C + v * 16, 16)
                        acc = ws[0] * rowsA[0, n, sl]
                        for k in range(1, KA):
                            acc = acc + ws[k] * rowsA[k, n, sl]
                        outb[n, sl] = acc
                else:
                    ws = [wbuf[b, k, pl.ds(nloc, 16)][0]
                          for k in range(KA, K)]
                    for v in range(V16):
                        sl = pl.ds(b * C + v * 16, 16)
                        acc = ws[0] * rowsB[0, n, sl]
                        for k in range(KA + 1, K):
                            acc = acc + ws[k - KA] * rowsB[k - KA, n, sl]
                        plsc.addupdate(outb.at[n, sl], acc)
                return 0

            lax.fori_loop(0, B, acc_batch, 0)

    fire(0, 0)
    fire(1, 0)

    def blk_body(blk, _):
        drain(0, blk)
        accum(0, blk)

        @pl.when(blk + 1 < NBLK)
        def _():
            fire(0, blk + 1)

        drain(1, blk)
        accum(1, blk)

        @pl.when(blk + 1 < NBLK)
        def _():
            fire(1, blk + 1)

        pltpu.sync_copy(outb, out_hbm.at[pl.ds(node_base + blk * NB, NB)])
        return 0

    lax.fori_loop(0, NBLK, blk_body, 0)


def _sc_pool(xt, idx_w, p):
    mesh = plsc.VectorSubcoreMesh(core_axis_name="c", subcore_axis_name="s")
    kern = functools.partial(
        pl.kernel,
        mesh=mesh,
        compiler_params=pltpu.CompilerParams(
            needs_layout_passes=False, use_tc_tiling_on_sc=False),
        out_type=jax.ShapeDtypeStruct((N, B * C), jnp.float32),
        scratch_types=[
            pltpu.VMEM((CHUNK, K), jnp.int32),
            pltpu.VMEM((K, CHUNK), jnp.int32),
            pltpu.VMEM((N,), jnp.float32),
            pltpu.VMEM((B, K, CHUNK + 16), jnp.float32),
            pltpu.VMEM((KA, NB, B * C), jnp.float32),
            pltpu.VMEM((K - KA, NB, B * C), jnp.float32),
            pltpu.VMEM((NB, B * C), jnp.float32),
            pltpu.SemaphoreType.DMA,
            pltpu.SemaphoreType.DMA,
        ],
    )(_sc_pool_body)
    return kern(xt, idx_w, p)


def _gn_body(y_ref, g_ref, b_ref, o_ref):
    y = y_ref[...]
    s_ch = jnp.sum(y, axis=0, keepdims=True)
    q_ch = jnp.sum(y * y, axis=0, keepdims=True)
    gi = lax.broadcasted_iota(jnp.int32, (C, C), 0) // (C // G)
    gj = lax.broadcasted_iota(jnp.int32, (C, C), 1) // (C // G)
    M = jnp.where(gi == gj, 1.0 / ((C // G) * N), 0.0).astype(jnp.float32)
    mean_c = jnp.dot(s_ch, M, preferred_element_type=jnp.float32)
    ex2_c = jnp.dot(q_ch, M, preferred_element_type=jnp.float32)
    var_c = ex2_c - mean_c * mean_c
    rstd_c = lax.rsqrt(var_c + EPS)
    gam = g_ref[...].reshape(1, C)
    bet = b_ref[...].reshape(1, C)
    o_ref[...] = ((y - mean_c) * (rstd_c * gam) + bet).reshape(1, N, C)


def _group_norm(pool_t, gamma, beta):
    return pl.pallas_call(
        _gn_body,
        grid=(B,),
        in_specs=[
            pl.BlockSpec((N, C), lambda b: (0, b)),
            pl.BlockSpec((C,), lambda b: (0,)),
            pl.BlockSpec((C,), lambda b: (0,)),
        ],
        out_specs=pl.BlockSpec((1, N, C), lambda b: (b, 0, 0)),
        out_shape=jax.ShapeDtypeStruct((B, N, C), jnp.float32),
    )(pool_t, gamma, beta)


def kernel(x, dynamic_indices, ro_W, ro_b, gamma, beta):
    p3, xt = _project(x, ro_W, ro_b)
    p = p3.reshape(B, N)
    pool_t = _sc_pool(xt, dynamic_indices, p)
    return _group_norm(pool_t, gamma, beta)
